# baseline (speedup 1.0000x reference)
import jax
import jax.numpy as jnp
from jax import lax
from jax.experimental import pallas as pl
from jax.experimental.pallas import tpu as pltpu
from jax.experimental.pallas import tpu_sc as plsc

_NC = 2
_NS = 16
_C = 80
_NBUF = 3
_NPH = 4


def _lane_bcast(vec, j):
    idx = jnp.full((16, 1), j, jnp.int32)
    dnums = lax.GatherDimensionNumbers(
        offset_dims=(), collapsed_slice_dims=(0,), start_index_map=(0,))
    return lax.gather(vec, idx, dnums, slice_sizes=(1,),
                      mode=lax.GatherScatterMode.PROMISE_IN_BOUNDS)


def _tc_body(p_ref, w_ref, b_ref, a_ref, o_ref):
    x = p_ref[0] + p_ref[1]
    s = jnp.dot(x, w_ref[...], preferred_element_type=jnp.float32)
    s = s + b_ref[...]
    a = a_ref[0, 0]
    o_ref[...] = jnp.where(s >= 0, s, a * s)


def _sc_spmm(x, edge_index, edge_weight, n, f):
    e = edge_weight.shape[0]
    nw = _NC * _NS
    ept = e // nw
    nstep = ept // _C
    nst0 = -(-nstep // _NPH)
    base_rows = (n // _NS) // 8 * 8
    tail_rows = n - _NS * base_rows
    zrows = _NBUF * _C
    assert tail_rows % 8 == 0 and tail_rows <= zrows
    assert base_rows > 2 * zrows and base_rows - 2 * zrows <= zrows

    mesh = plsc.VectorSubcoreMesh(core_axis_name="c", subcore_axis_name="s")

    def body(x_hbm, src_hbm, dst_hbm, ew_hbm, out_hbm,
             src_a, src_b, dst_a, dst_b, w_a, w_b, dst_dbA, dst_dbB,
             rows_db, acc,
             g0, g1, g2, s0, s1, s2, stg0, stg1):
        cid = lax.axis_index("c")
        sid = lax.axis_index("s")
        wid = cid * _NS + sid
        gsems = (g0, g1, g2)
        ssems = (s0, s1, s2)
        stgsems = (stg0, stg1)
        src_bufs = (src_a, src_b)
        dst_bufs = (dst_a, dst_b)
        w_bufs = (w_a, w_b)
        row0 = sid * base_rows
        ebase = wid * ept

        def rows(p):
            return rows_db.at[pl.ds(p * _C, _C), :]

        _HA = 48
        _HB = _C - _HA
        _HOFF = (0, _HA)
        _HLEN = (_HA, _HB)

        def halfrows(p, u):
            return rows_db.at[pl.ds(p * _C + _HOFF[u], _HLEN[u]), :]

        def half_idx(p, u):
            return dst_dbA.at[p] if u == 0 else dst_dbB.at[p]

        def issue_gather(s, p, b):
            for k in range(_HA // 16):
                sl16 = pl.ds(k * 16, 16)
                dst_dbA[p, sl16] = dst_bufs[b][pl.ds(s * _C + k * 16, 16)]
            for k in range(_HB // 16):
                sl16 = pl.ds(k * 16, 16)
                dst_dbB[p, sl16] = (
                    dst_bufs[b][pl.ds(s * _C + _HA + k * 16, 16)])
            for u in range(2):
                pltpu.async_copy(
                    x_hbm.at[src_bufs[b].at[
                        pl.ds(s * _C + _HOFF[u], _HLEN[u])]],
                    halfrows(p, u), gsems[p])

        def wait_gather(p):
            for u in range(2):
                pltpu.make_async_copy(
                    x_hbm.at[src_a.at[pl.ds(0, _HLEN[u])]], halfrows(p, u),
                    gsems[p]).wait()

        def issue_scatter(p):
            for u in range(2):
                pltpu.async_copy(halfrows(p, u), acc.at[half_idx(p, u)],
                                 ssems[p], add=True)

        def wait_scatter(p):
            for u in range(2):
                pltpu.make_async_copy(halfrows(p, u),
                                      acc.at[half_idx(p, u)],
                                      ssems[p]).wait()

        def scale(s, p, b):
            @pl.loop(0, _C // 16)
            def _scale(i16):
                e0 = i16 * 16
                wvec = w_bufs[b][pl.ds(s * _C + e0, 16)]
                for j in range(16):
                    wb = _lane_bcast(wvec, j)
                    for k in range(f // 16):
                        sl = pl.ds(k * 16, 16)
                        r = p * _C + e0 + j
                        rows_db[r, sl] = rows_db[r, sl] * wb

        def phase_len(h):
            return min(nst0, nstep - h * nst0)

        def stage_issue(h):
            b = h % 2
            hbase = ebase + h * nst0 * _C
            ne = phase_len(h) * _C
            pltpu.async_copy(src_hbm.at[pl.ds(hbase, ne)],
                             src_bufs[b].at[pl.ds(0, ne)], stgsems[b])
            pltpu.async_copy(dst_hbm.at[pl.ds(hbase, ne)],
                             dst_bufs[b].at[pl.ds(0, ne)], stgsems[b])
            pltpu.async_copy(ew_hbm.at[pl.ds(hbase, ne)],
                             w_bufs[b].at[pl.ds(0, ne)], stgsems[b])

        def stage_wait(h):
            b = h % 2
            ne = phase_len(h) * _C
            pltpu.make_async_copy(src_hbm.at[pl.ds(0, ne)],
                                  src_bufs[b].at[pl.ds(0, ne)],
                                  stgsems[b]).wait()
            pltpu.make_async_copy(dst_hbm.at[pl.ds(0, ne)],
                                  dst_bufs[b].at[pl.ds(0, ne)],
                                  stgsems[b]).wait()
            pltpu.make_async_copy(ew_hbm.at[pl.ds(0, ne)],
                                  w_bufs[b].at[pl.ds(0, ne)],
                                  stgsems[b]).wait()

        stage_issue(0)
        zeros16 = jnp.zeros((16,), jnp.float32)

        @pl.loop(0, zrows)
        def _zero_rows(i):
            for k in range(f // 16):
                rows_db[i, pl.ds(k * 16, 16)] = zeros16

        zsizes = (zrows, zrows, base_rows - 2 * zrows)
        off = 0
        for rcount in zsizes:
            pltpu.async_copy(rows_db.at[pl.ds(0, rcount), :],
                             acc.at[pl.ds(row0 + off, rcount), :], g0)
            off += rcount

        @pl.when(sid == _NS - 1)
        def _zero_tail():
            pltpu.async_copy(rows_db.at[pl.ds(0, tail_rows), :],
                            acc.at[pl.ds(_NS * base_rows, tail_rows), :], g1)

        for rcount in zsizes:
            pltpu.make_async_copy(rows_db.at[pl.ds(0, rcount), :],
                                  acc.at[pl.ds(row0, rcount), :], g0).wait()

        @pl.when(sid == _NS - 1)
        def _zero_tail_wait():
            pltpu.make_async_copy(
                rows_db.at[pl.ds(0, tail_rows), :],
                acc.at[pl.ds(_NS * base_rows, tail_rows), :], g1).wait()

        stage_wait(0)
        plsc.subcore_barrier()

        def run_phase(h):
            b = h % 2
            nst = phase_len(h)
            t_last = nst - 1
            for s in range(min(_NBUF, nst)):
                issue_gather(s, s, b)
            wait_gather(0)
            scale(0, 0, b)
            issue_scatter(0)
            ntr = max(0, (t_last - 4) // 3)
            if ntr:
                @pl.loop(0, ntr)
                def _triples(i):
                    sbase = 1 + 3 * i
                    for d in range(3):
                        p = (1 + d) % 3
                        q = (d + 3) % 3
                        wait_scatter(q)
                        issue_gather(sbase + d + 2, q, b)
                        wait_gather(p)
                        scale(sbase + d, p, b)
                        issue_scatter(p)
            for s in range(3 * ntr + 1, t_last + 1):
                p = s % 3
                if s + 2 <= t_last:
                    q = (s + 2) % 3
                    wait_scatter(q)
                    issue_gather(s + 2, q, b)
                wait_gather(p)
                scale(s, p, b)
                issue_scatter(p)
            for s in range(max(0, t_last - 2), t_last + 1):
                wait_scatter(s % 3)

        for h in range(_NPH):
            if h + 1 < _NPH:
                stage_issue(h + 1)
            run_phase(h)
            if h + 1 < _NPH:
                stage_wait(h + 1)

        plsc.subcore_barrier()
        pltpu.sync_copy(acc.at[pl.ds(row0, base_rows), :],
                        out_hbm.at[cid, pl.ds(row0, base_rows), :])

        @pl.when(sid == _NS - 1)
        def _copy_tail():
            t0 = _NS * base_rows
            pltpu.sync_copy(acc.at[pl.ds(t0, tail_rows), :],
                            out_hbm.at[cid, pl.ds(t0, tail_rows), :])

    run = pl.kernel(
        body,
        out_type=jax.ShapeDtypeStruct((_NC, n, f), jnp.float32),
        mesh=mesh,
        scratch_types=[
            pltpu.VMEM((nst0 * _C,), jnp.int32),
            pltpu.VMEM((nst0 * _C,), jnp.int32),
            pltpu.VMEM((nst0 * _C,), jnp.int32),
            pltpu.VMEM((nst0 * _C,), jnp.int32),
            pltpu.VMEM((nst0 * _C,), jnp.float32),
            pltpu.VMEM((nst0 * _C,), jnp.float32),
            pltpu.VMEM((_NBUF, 48), jnp.int32),
            pltpu.VMEM((_NBUF, _C - 48), jnp.int32),
            pltpu.VMEM((_NBUF * _C, f), jnp.float32),
            pltpu.VMEM_SHARED((n, f), jnp.float32),
            pltpu.SemaphoreType.DMA,
            pltpu.SemaphoreType.DMA,
            pltpu.SemaphoreType.DMA,
            pltpu.SemaphoreType.DMA,
            pltpu.SemaphoreType.DMA,
            pltpu.SemaphoreType.DMA,
            pltpu.SemaphoreType.DMA,
            pltpu.SemaphoreType.DMA,
        ],
    )
    return run(x, edge_index[0], edge_index[1], edge_weight)


def kernel(X, edge_index, edge_weight, W, b, prelu_a):
    n, f_in = X.shape
    f = W.shape[1]
    bm = 1000

    partials = _sc_spmm(X, edge_index, edge_weight, n, f_in)

    b2 = b.reshape(1, f)
    a2 = prelu_a.reshape(1, 1)
    out = pl.pallas_call(
        _tc_body,
        grid=(n // bm,),
        in_specs=[
            pl.BlockSpec((_NC, bm, f_in), lambda i: (0, i, 0)),
            pl.BlockSpec((f_in, f), lambda i: (0, 0)),
            pl.BlockSpec((1, f), lambda i: (0, 0)),
            pl.BlockSpec((1, 1), lambda i: (0, 0)),
        ],
        out_specs=pl.BlockSpec((bm, f), lambda i: (i, 0)),
        out_shape=jax.ShapeDtypeStruct((n, f), jnp.float32),
    )(partials, W, b2, a2)
    return out


# speedup vs baseline: 1.0049x; 1.0049x over previous
import jax
import jax.numpy as jnp
from jax import lax
from jax.experimental import pallas as pl
from jax.experimental.pallas import tpu as pltpu
from jax.experimental.pallas import tpu_sc as plsc

_NC = 2
_NS = 16
_C = 80
_NBUF = 3
_NPH = 4


def _lane_bcast(vec, j):
    idx = jnp.full((16, 1), j, jnp.int32)
    dnums = lax.GatherDimensionNumbers(
        offset_dims=(), collapsed_slice_dims=(0,), start_index_map=(0,))
    return lax.gather(vec, idx, dnums, slice_sizes=(1,),
                      mode=lax.GatherScatterMode.PROMISE_IN_BOUNDS)


def _tc_body(p_ref, w_ref, b_ref, a_ref, o_ref):
    x = p_ref[0] + p_ref[1]
    s = jnp.dot(x, w_ref[...], preferred_element_type=jnp.float32)
    s = s + b_ref[...]
    a = a_ref[0, 0]
    o_ref[...] = jnp.where(s >= 0, s, a * s)


def _sc_spmm(x, edge_index, edge_weight, n, f):
    e = edge_weight.shape[0]
    nw = _NC * _NS
    ept = e // nw
    nstep = ept // _C
    nst0 = -(-nstep // _NPH)
    base_rows = (n // _NS) // 8 * 8
    tail_rows = n - _NS * base_rows
    zrows = _NBUF * _C
    assert tail_rows % 8 == 0 and tail_rows <= zrows
    assert base_rows > 2 * zrows and base_rows - 2 * zrows <= zrows

    mesh = plsc.VectorSubcoreMesh(core_axis_name="c", subcore_axis_name="s")

    def body(x_hbm, src_hbm, dst_hbm, ew_hbm, out_hbm,
             src_a, src_b, dst_a, dst_b, w_a, w_b, dst_db, rows_db, acc,
             g0, g1, g2, s0, s1, s2, stg0, stg1):
        cid = lax.axis_index("c")
        sid = lax.axis_index("s")
        wid = cid * _NS + sid
        gsems = (g0, g1, g2)
        ssems = (s0, s1, s2)
        stgsems = (stg0, stg1)
        src_bufs = (src_a, src_b)
        dst_bufs = (dst_a, dst_b)
        w_bufs = (w_a, w_b)
        row0 = sid * base_rows
        ebase = wid * ept

        def rows(p):
            return rows_db.at[pl.ds(p * _C, _C), :]

        def issue_gather(s, p, b):
            for k in range(_C // 16):
                sl = pl.ds(k * 16, 16)
                dst_db[p, sl] = dst_bufs[b][pl.ds(s * _C + k * 16, 16)]
            pltpu.async_copy(x_hbm.at[src_bufs[b].at[pl.ds(s * _C, _C)]],
                             rows(p), gsems[p])

        def wait_gather(p):
            pltpu.make_async_copy(
                x_hbm.at[src_a.at[pl.ds(0, _C)]], rows(p),
                gsems[p]).wait()

        def issue_scatter(p):
            pltpu.async_copy(rows(p), acc.at[dst_db.at[p]], ssems[p],
                             add=True)

        def wait_scatter(p):
            pltpu.make_async_copy(rows(p), acc.at[dst_db.at[p]],
                                  ssems[p]).wait()

        def scale(s, p, b):
            @pl.loop(0, _C // 16)
            def _scale(i16):
                e0 = i16 * 16
                wvec = w_bufs[b][pl.ds(s * _C + e0, 16)]
                for j in range(16):
                    wb = _lane_bcast(wvec, j)
                    for k in range(f // 16):
                        sl = pl.ds(k * 16, 16)
                        r = p * _C + e0 + j
                        rows_db[r, sl] = rows_db[r, sl] * wb

        def phase_len(h):
            return min(nst0, nstep - h * nst0)

        def stage_issue(h):
            b = h % 2
            hbase = ebase + h * nst0 * _C
            ne = phase_len(h) * _C
            pltpu.async_copy(src_hbm.at[pl.ds(hbase, ne)],
                             src_bufs[b].at[pl.ds(0, ne)], stgsems[b])
            pltpu.async_copy(dst_hbm.at[pl.ds(hbase, ne)],
                             dst_bufs[b].at[pl.ds(0, ne)], stgsems[b])
            pltpu.async_copy(ew_hbm.at[pl.ds(hbase, ne)],
                             w_bufs[b].at[pl.ds(0, ne)], stgsems[b])

        def stage_wait(h):
            b = h % 2
            ne = phase_len(h) * _C
            pltpu.make_async_copy(src_hbm.at[pl.ds(0, ne)],
                                  src_bufs[b].at[pl.ds(0, ne)],
                                  stgsems[b]).wait()
            pltpu.make_async_copy(dst_hbm.at[pl.ds(0, ne)],
                                  dst_bufs[b].at[pl.ds(0, ne)],
                                  stgsems[b]).wait()
            pltpu.make_async_copy(ew_hbm.at[pl.ds(0, ne)],
                                  w_bufs[b].at[pl.ds(0, ne)],
                                  stgsems[b]).wait()

        stage_issue(0)
        zeros16 = jnp.zeros((16,), jnp.float32)

        @pl.loop(0, zrows)
        def _zero_rows(i):
            for k in range(f // 16):
                rows_db[i, pl.ds(k * 16, 16)] = zeros16

        zsizes = (zrows, zrows, base_rows - 2 * zrows)
        off = 0
        for rcount in zsizes:
            pltpu.async_copy(rows_db.at[pl.ds(0, rcount), :],
                             acc.at[pl.ds(row0 + off, rcount), :], g0)
            off += rcount

        @pl.when(sid == _NS - 1)
        def _zero_tail():
            pltpu.async_copy(rows_db.at[pl.ds(0, tail_rows), :],
                            acc.at[pl.ds(_NS * base_rows, tail_rows), :], g1)

        for rcount in zsizes:
            pltpu.make_async_copy(rows_db.at[pl.ds(0, rcount), :],
                                  acc.at[pl.ds(row0, rcount), :], g0).wait()

        @pl.when(sid == _NS - 1)
        def _zero_tail_wait():
            pltpu.make_async_copy(
                rows_db.at[pl.ds(0, tail_rows), :],
                acc.at[pl.ds(_NS * base_rows, tail_rows), :], g1).wait()

        stage_wait(0)
        plsc.subcore_barrier()

        def run_phase(h):
            b = h % 2
            nst = phase_len(h)
            t_last = nst - 1
            for s in range(min(_NBUF, nst)):
                issue_gather(s, s, b)
            wait_gather(0)
            scale(0, 0, b)
            issue_scatter(0)
            ntr = max(0, (t_last - 4) // 3)
            if ntr:
                @pl.loop(0, ntr)
                def _triples(i):
                    sbase = 1 + 3 * i
                    for d in range(3):
                        p = (1 + d) % 3
                        q = (d + 3) % 3
                        wait_scatter(q)
                        issue_gather(sbase + d + 2, q, b)
                        wait_gather(p)
                        scale(sbase + d, p, b)
                        issue_scatter(p)
            for s in range(3 * ntr + 1, t_last + 1):
                p = s % 3
                if s + 2 <= t_last:
                    q = (s + 2) % 3
                    wait_scatter(q)
                    issue_gather(s + 2, q, b)
                wait_gather(p)
                scale(s, p, b)
                issue_scatter(p)
            for s in range(max(0, t_last - 2), t_last + 1):
                wait_scatter(s % 3)

        for h in range(_NPH):
            if h + 1 < _NPH:
                stage_issue(h + 1)
            run_phase(h)
            if h + 1 < _NPH:
                stage_wait(h + 1)

        plsc.subcore_barrier()
        pltpu.sync_copy(acc.at[pl.ds(row0, base_rows), :],
                        out_hbm.at[cid, pl.ds(row0, base_rows), :])

        @pl.when(sid == _NS - 1)
        def _copy_tail():
            t0 = _NS * base_rows
            pltpu.sync_copy(acc.at[pl.ds(t0, tail_rows), :],
                            out_hbm.at[cid, pl.ds(t0, tail_rows), :])

    run = pl.kernel(
        body,
        out_type=jax.ShapeDtypeStruct((_NC, n, f), jnp.float32),
        mesh=mesh,
        scratch_types=[
            pltpu.VMEM((nst0 * _C,), jnp.int32),
            pltpu.VMEM((nst0 * _C,), jnp.int32),
            pltpu.VMEM((nst0 * _C,), jnp.int32),
            pltpu.VMEM((nst0 * _C,), jnp.int32),
            pltpu.VMEM((nst0 * _C,), jnp.float32),
            pltpu.VMEM((nst0 * _C,), jnp.float32),
            pltpu.VMEM((_NBUF, _C), jnp.int32),
            pltpu.VMEM((_NBUF * _C, f), jnp.float32),
            pltpu.VMEM_SHARED((n, f), jnp.float32),
            pltpu.SemaphoreType.DMA,
            pltpu.SemaphoreType.DMA,
            pltpu.SemaphoreType.DMA,
            pltpu.SemaphoreType.DMA,
            pltpu.SemaphoreType.DMA,
            pltpu.SemaphoreType.DMA,
            pltpu.SemaphoreType.DMA,
            pltpu.SemaphoreType.DMA,
        ],
    )
    return run(x, edge_index[0], edge_index[1], edge_weight)


def kernel(X, edge_index, edge_weight, W, b, prelu_a):
    n, f_in = X.shape
    f = W.shape[1]
    bm = 1000

    partials = _sc_spmm(X, edge_index, edge_weight, n, f_in)

    b2 = b.reshape(1, f)
    a2 = prelu_a.reshape(1, 1)
    out = pl.pallas_call(
        _tc_body,
        grid=(n // bm,),
        in_specs=[
            pl.BlockSpec((_NC, bm, f_in), lambda i: (0, i, 0)),
            pl.BlockSpec((f_in, f), lambda i: (0, 0)),
            pl.BlockSpec((1, f), lambda i: (0, 0)),
            pl.BlockSpec((1, 1), lambda i: (0, 0)),
        ],
        out_specs=pl.BlockSpec((bm, f), lambda i: (i, 0)),
        out_shape=jax.ShapeDtypeStruct((n, f), jnp.float32),
    )(partials, W, b2, a2)
    return out
